# baseline (speedup 1.0000x reference)
import functools

import numpy as np
import jax
import jax.numpy as jnp
from jax import lax
from jax.experimental import pallas as pl
from jax.experimental.pallas import tpu as pltpu
from jax.experimental.pallas import tpu_sc as plsc

EMBED = 512
HID = 128
BLK = 2048
BETA = 0.25
SQRT_HALF = np.float32(np.sqrt(0.5))


def _erfc(x):
    one = jnp.float32(1.0)
    x2 = x * x
    t = x2 * jnp.float32(7.85386146e-05)
    for c in (-0.000801019371, 0.00518832775, -0.0268538129,
              0.112835854, -0.37612626, 1.12837911):
        t = t + jnp.float32(c)
        if c != 1.12837911:
            t = t * x2
    small = one - x * t
    abs_x = jnp.abs(x)
    q = one / abs_x
    ez = jnp.exp(-x2)
    zq = ez * q
    yy = one / x2
    pP = yy * jnp.float32(0.0232682)
    for c in (-0.138703942, 0.368742466, -0.582473278, 0.621000469,
              -0.494451523, 0.340488, -0.274112701, 0.563825965):
        pP = pP + jnp.float32(c)
        if c != 0.563825965:
            pP = pP * yy
    pR = yy * jnp.float32(-10.477664)
    for c in (12.9772, -7.49551868, 2.92101908, -1.01526523,
              0.42184633, -0.282076746, 0.564189494):
        pR = pR + jnp.float32(c)
        if c != 0.564189494:
            pR = pR * yy
    p = jnp.where(abs_x < 2.0, pP, pR)
    yb = zq * p
    yb = jnp.where(-x2 < jnp.float32(-88.7228394), jnp.float32(0.0), yb)
    yb = jnp.where(x < 0.0, jnp.float32(2.0) - yb, yb)
    return jnp.where(abs_x < 1.0, small, yb)


def _gelu(x):
    return 0.5 * x * _erfc(-x * SQRT_HALF)


def _attn_combine_body(sa_ref, m2_ref, m3_ref, out_ref):
    def norm(x):
        mn = jnp.min(x, axis=-1, keepdims=True)
        mx = jnp.max(x, axis=-1, keepdims=True)
        return (x - mn) / (mx - mn + 1e-08)

    s_im = norm(sa_ref[...])
    s_m2 = norm(m2_ref[...])
    s_m3 = norm(m3_ref[...])
    out_ref[...] = (s_m2 + s_m3) + 2 * s_im


def _mlp_body(x_ref, w1_ref, b1_ref, w2_ref, b2_ref, out_ref):
    x = x_ref[0]
    h = _gelu(jnp.dot(x, w1_ref[...]) + b1_ref[...])
    p = jnp.dot(h, w2_ref[...]) + b2_ref[...]
    out_ref[0] = jax.nn.sigmoid(p)


def _sc_sort(ikey):
    B, N = ikey.shape
    NSEG = N // 16
    NK = N // 2
    mesh = plsc.VectorSubcoreMesh(core_axis_name="c", subcore_axis_name="s")

    @functools.partial(
        pl.kernel,
        out_type=[
            jax.ShapeDtypeStruct((B, N), jnp.int32),
            jax.ShapeDtypeStruct((B, N), jnp.int32),
            jax.ShapeDtypeStruct((B, N), jnp.float32),
        ],
        mesh=mesh,
        scratch_types=[
            pltpu.VMEM((N,), jnp.float32),
            pltpu.VMEM((N,), jnp.int32),
            pltpu.VMEM((N,), jnp.int32),
            pltpu.VMEM((N,), jnp.int32),
            pltpu.VMEM((N,), jnp.int32),
            pltpu.VMEM((4096,), jnp.int32),
            pltpu.VMEM((4096,), jnp.int32),
            pltpu.VMEM((256,), jnp.int32),
        ],
        compiler_params=pltpu.CompilerParams(needs_layout_passes=False),
    )
    def sortk(ikey_hbm, order_hbm, skey_hbm, mask_hbm,
              maskv, ka, kb, va, vb, hist, base, startb):
        wid = lax.axis_index("s") * 2 + lax.axis_index("c")

        @pl.when(wid < B)
        def _():
            row = wid
            pltpu.sync_copy(ikey_hbm.at[row], ka)
            lane = jnp.arange(16, dtype=jnp.int32)
            lane_seg = lane * NSEG
            lane256 = lane * 256
            ones_i = jnp.ones((16,), jnp.int32)

            for p in range(4):
                kc, vc, kn, vn = (ka, va, kb, vb) if p % 2 == 0 else (kb, vb, ka, va)
                shift = p * 8

                def zh(i, _):
                    hist[pl.ds(i * 16, 16)] = jnp.zeros((16,), jnp.int32)
                    return 0

                lax.fori_loop(0, 256, zh, 0)

                def cnt(g, _, kc=kc, shift=shift):
                    gi = lane_seg + g
                    k = plsc.load_gather(kc, [gi])
                    d = lax.shift_right_logical(k, shift) & 255
                    plsc.addupdate_scatter(hist, [lane256 + d], ones_i)
                    return 0

                lax.fori_loop(0, NSEG, cnt, 0)

                def b1(c, carry):
                    def acc_l(l, a):
                        return a + hist[pl.ds(l * 256 + c * 16, 16)]

                    tot = lax.fori_loop(0, 16, acc_l, jnp.zeros((16,), jnp.int32))
                    incl = plsc.cumsum(tot)
                    startb[pl.ds(c * 16, 16)] = incl - tot + carry
                    return carry + jnp.sum(tot)

                lax.fori_loop(0, 16, b1, jnp.int32(0))

                def b2(c, _):
                    def bl(l, run):
                        sl2 = pl.ds(l * 256 + c * 16, 16)
                        base[sl2] = run
                        return run + hist[sl2]

                    lax.fori_loop(0, 16, bl, startb[pl.ds(c * 16, 16)])
                    return 0

                lax.fori_loop(0, 16, b2, 0)

                def perm(g, _, p=p, kc=kc, vc=vc, kn=kn, vn=vn, shift=shift):
                    gi = lane_seg + g
                    k = plsc.load_gather(kc, [gi])
                    v = gi if p == 0 else plsc.load_gather(vc, [gi])
                    d = lax.shift_right_logical(k, shift) & 255
                    hidx = lane256 + d
                    pos = plsc.load_gather(base, [hidx])
                    plsc.store_scatter(base, [hidx], pos + ones_i)
                    plsc.store_scatter(kn, [pos], k)
                    plsc.store_scatter(vn, [pos], v)
                    return 0

                lax.fori_loop(0, NSEG, perm, 0)

            pltpu.sync_copy(va, order_hbm.at[row])
            pltpu.sync_copy(ka, skey_hbm.at[row])

            def zm(g, _):
                maskv[pl.ds(g * 16, 16)] = jnp.zeros((16,), jnp.float32)
                return 0

            lax.fori_loop(0, N // 16, zm, 0)
            ones_f = jnp.ones((16,), jnp.float32)

            def sm(g, _):
                idxv = va[pl.ds(g * 16, 16)]
                plsc.store_scatter(maskv, [idxv], ones_f)
                return 0

            lax.fori_loop(0, NK // 16, sm, 0)
            pltpu.sync_copy(maskv, mask_hbm.at[row])

    return sortk(ikey)


def _sc_select_gather(tokens_flat, order_flat, B, N):
    BN, C = tokens_flat.shape
    NK = N // 2
    NKW = NK // 4
    CHUNK = 64
    mesh = plsc.VectorSubcoreMesh(core_axis_name="c", subcore_axis_name="s")

    @functools.partial(
        pl.kernel,
        out_type=jax.ShapeDtypeStruct((B * NK, C), jnp.float32),
        mesh=mesh,
        scratch_types=[
            pltpu.VMEM((NKW,), jnp.int32),
            pltpu.VMEM((CHUNK, C), jnp.float32),
            pltpu.VMEM((CHUNK, C), jnp.float32),
            pltpu.SemaphoreType.DMA,
            pltpu.SemaphoreType.DMA,
        ],
    )
    def gk(tok_hbm, ord_hbm, out_hbm, idxv, buf0, buf1, sem0, sem1):
        wid = lax.axis_index("s") * 2 + lax.axis_index("c")
        b = wid // 4
        qq = wid % 4
        src_off = b * N + qq * NKW
        dst_off = b * NK + qq * NKW
        pltpu.sync_copy(ord_hbm.at[pl.ds(src_off, NKW)], idxv)
        badd = b * N

        def addb(i, _):
            sl = pl.ds(i * 16, 16)
            idxv[sl] = idxv[sl] + badd
            return 0

        lax.fori_loop(0, NKW // 16, addb, 0)
        bufs = (buf0, buf1)
        sems = (sem0, sem1)

        def start(c):
            return pltpu.async_copy(
                tok_hbm.at[idxv.at[pl.ds(c * CHUNK, CHUNK)]],
                bufs[c % 2], sems[c % 2])

        nch = NKW // CHUNK
        handles = [start(0), None]
        for c in range(nch):
            if c + 1 < nch:
                handles[(c + 1) % 2] = start(c + 1)
            handles[c % 2].wait()
            pltpu.sync_copy(bufs[c % 2],
                            out_hbm.at[pl.ds(dst_off + c * CHUNK, CHUNK)])

    return gk(tokens_flat, order_flat)


def _extra_body(x_ref, k_ref, m_ref, mxk_ref, extra_ref, acc_ref, z_ref):
    kk = k_ref[0, 0]
    ukey = ~kk
    u = jnp.where(ukey < 0, ukey ^ jnp.int32(-2147483648), ~ukey)
    s = lax.bitcast_convert_type(u, jnp.float32)
    mk = ~mxk_ref[0]
    umx = jnp.where(mk < 0, mk ^ jnp.int32(-2147483648), ~mk)
    mx = lax.bitcast_convert_type(umx, jnp.float32)
    e = jnp.exp(s - mx)
    w = jnp.where(m_ref[0, 0] > 0, jnp.float32(0.0), e)
    blk = w.shape[1]
    seg = jnp.arange(blk, dtype=jnp.int32)[None, :] // (blk // 8)
    w8 = jnp.where(seg == jnp.arange(8, dtype=jnp.int32)[:, None], w, jnp.float32(0.0))
    p8 = jnp.dot(w8, x_ref[0])
    p = jnp.sum(p8, axis=0, keepdims=True)
    zp = jnp.sum(w, axis=1, keepdims=True)
    n = pl.program_id(1)

    @pl.when(n == 0)
    def _():
        acc_ref[...] = jnp.zeros_like(acc_ref)
        z_ref[...] = jnp.zeros_like(z_ref)

    acc = acc_ref[...] + p
    z = z_ref[...] + zp
    acc_ref[...] = acc
    z_ref[...] = z

    @pl.when(n == pl.num_programs(1) - 1)
    def _():
        extra_ref[0] = acc / z


def _keymake_body(sp_ref, ac_ref, out_ref):
    score = jnp.float32(0.5) * sp_ref[...] + jnp.float32(0.25) * ac_ref[...]
    u = lax.bitcast_convert_type(score, jnp.int32)
    m32 = (u >> 31) | jnp.int32(-2147483648)
    out_ref[...] = ~(u ^ m32)


def _keyinv_body(k_ref, out_ref):
    ukey = ~k_ref[...]
    u = jnp.where(ukey < 0, ukey ^ jnp.int32(-2147483648), ~ukey)
    out_ref[...] = lax.bitcast_convert_type(u, jnp.float32)


def kernel(tokens, self_attention, cross_attention_m2, cross_attention_m3, W1, b1, W2, b2):
    B, N, C = tokens.shape
    num_keep = N // 2

    acomb = pl.pallas_call(
        _attn_combine_body,
        out_shape=jax.ShapeDtypeStruct((B, N), jnp.float32),
    )(self_attention, cross_attention_m2, cross_attention_m3)

    s_pred3 = pl.pallas_call(
        _mlp_body,
        grid=(B, N // BLK),
        in_specs=[
            pl.BlockSpec((1, BLK, EMBED), lambda b, n: (b, n, 0)),
            pl.BlockSpec((EMBED, HID), lambda b, n: (0, 0)),
            pl.BlockSpec((1, HID), lambda b, n: (0, 0)),
            pl.BlockSpec((HID, 1), lambda b, n: (0, 0)),
            pl.BlockSpec((1, 1), lambda b, n: (0, 0)),
        ],
        out_specs=pl.BlockSpec((1, BLK, 1), lambda b, n: (b, n, 0)),
        out_shape=jax.ShapeDtypeStruct((B, N, 1), jnp.float32),
    )(tokens, W1, b1.reshape(1, HID), W2, b2.reshape(1, 1))

    ikey = pl.pallas_call(
        _keymake_body,
        out_shape=jax.ShapeDtypeStruct((B, N), jnp.int32),
    )(s_pred3.reshape(B, N), acomb)
    order, skeys, score_mask = _sc_sort(ikey)
    keep_policy = order[:, :num_keep]

    select_flat = _sc_select_gather(tokens.reshape(B * N, C), order.reshape(B * N), B, N)
    select_tokens = select_flat.reshape(B, num_keep, C)

    extra = pl.pallas_call(
        _extra_body,
        grid=(B, N // BLK),
        in_specs=[
            pl.BlockSpec((1, BLK, EMBED), lambda b, n: (b, n, 0)),
            pl.BlockSpec((1, 1, 1, BLK), lambda b, n: (b, n, 0, 0)),
            pl.BlockSpec((1, 1, 1, BLK), lambda b, n: (b, n, 0, 0)),
            pl.BlockSpec((1, 1, 1), lambda b, n: (b, 0, 0)),
        ],
        out_specs=pl.BlockSpec((1, 1, EMBED), lambda b, n: (b, 0, 0)),
        out_shape=jax.ShapeDtypeStruct((B, 1, EMBED), jnp.float32),
        scratch_shapes=[
            pltpu.VMEM((1, EMBED), jnp.float32),
            pltpu.VMEM((1, 1), jnp.float32),
        ],
    )(tokens, ikey.reshape(B, N // BLK, 1, BLK), score_mask.reshape(B, N // BLK, 1, BLK),
      skeys[:, num_keep:num_keep + 1].reshape(B, 1, 1))
    extra_token = extra

    selected_mask = jnp.ones((B, num_keep), jnp.float32)
    return (select_tokens, extra_token, score_mask, selected_mask, keep_policy)


# speedup vs baseline: 1.0396x; 1.0396x over previous
import functools

import numpy as np
import jax
import jax.numpy as jnp
from jax import lax
from jax.experimental import pallas as pl
from jax.experimental.pallas import tpu as pltpu
from jax.experimental.pallas import tpu_sc as plsc

EMBED = 512
HID = 128
BLK = 2048
BETA = 0.25
SQRT_HALF = np.float32(np.sqrt(0.5))


def _erfc(x):
    one = jnp.float32(1.0)
    x2 = x * x
    t = x2 * jnp.float32(7.85386146e-05)
    for c in (-0.000801019371, 0.00518832775, -0.0268538129,
              0.112835854, -0.37612626, 1.12837911):
        t = t + jnp.float32(c)
        if c != 1.12837911:
            t = t * x2
    small = one - x * t
    abs_x = jnp.abs(x)
    q = one / abs_x
    ez = jnp.exp(-x2)
    zq = ez * q
    yy = one / x2
    pP = yy * jnp.float32(0.0232682)
    for c in (-0.138703942, 0.368742466, -0.582473278, 0.621000469,
              -0.494451523, 0.340488, -0.274112701, 0.563825965):
        pP = pP + jnp.float32(c)
        if c != 0.563825965:
            pP = pP * yy
    pR = yy * jnp.float32(-10.477664)
    for c in (12.9772, -7.49551868, 2.92101908, -1.01526523,
              0.42184633, -0.282076746, 0.564189494):
        pR = pR + jnp.float32(c)
        if c != 0.564189494:
            pR = pR * yy
    p = jnp.where(abs_x < 2.0, pP, pR)
    yb = zq * p
    yb = jnp.where(-x2 < jnp.float32(-88.7228394), jnp.float32(0.0), yb)
    yb = jnp.where(x < 0.0, jnp.float32(2.0) - yb, yb)
    return jnp.where(abs_x < 1.0, small, yb)


def _gelu(x):
    return 0.5 * x * _erfc(-x * SQRT_HALF)


def _attn_combine_body(sa_ref, m2_ref, m3_ref, out_ref):
    def norm(x):
        mn = jnp.min(x, axis=-1, keepdims=True)
        mx = jnp.max(x, axis=-1, keepdims=True)
        return (x - mn) / (mx - mn + 1e-08)

    s_im = norm(sa_ref[...])
    s_m2 = norm(m2_ref[...])
    s_m3 = norm(m3_ref[...])
    out_ref[...] = (s_m2 + s_m3) + 2 * s_im


def _mlp_body(x_ref, w1_ref, b1_ref, w2_ref, b2_ref, out_ref):
    x = x_ref[0]
    h = _gelu(jnp.dot(x, w1_ref[...]) + b1_ref[...])
    p = jnp.dot(h, w2_ref[...]) + b2_ref[...]
    out_ref[0] = jax.nn.sigmoid(p)


def _sc_sort(ikey):
    B, N = ikey.shape
    NSEG = N // 16
    NK = N // 2
    mesh = plsc.VectorSubcoreMesh(core_axis_name="c", subcore_axis_name="s")

    @functools.partial(
        pl.kernel,
        out_type=[
            jax.ShapeDtypeStruct((B, N), jnp.int32),
            jax.ShapeDtypeStruct((B, N), jnp.int32),
            jax.ShapeDtypeStruct((B, N), jnp.float32),
        ],
        mesh=mesh,
        scratch_types=[
            pltpu.VMEM((N,), jnp.float32),
            pltpu.VMEM((N,), jnp.int32),
            pltpu.VMEM((N,), jnp.int32),
            pltpu.VMEM((N,), jnp.int32),
            pltpu.VMEM((N,), jnp.int32),
            pltpu.VMEM((8192,), jnp.int32),
            pltpu.VMEM((8192,), jnp.int32),
            pltpu.VMEM((256,), jnp.int32),
        ],
        compiler_params=pltpu.CompilerParams(needs_layout_passes=False),
    )
    def sortk(ikey_hbm, order_hbm, skey_hbm, mask_hbm,
              maskv, ka, kb, va, vb, hist, base, startb):
        wid = lax.axis_index("s") * 2 + lax.axis_index("c")

        @pl.when(wid < B)
        def _():
            row = wid
            pltpu.sync_copy(ikey_hbm.at[row], ka)
            lane = jnp.arange(16, dtype=jnp.int32)
            lane_seg = lane * NSEG
            vt_a = lane * NSEG
            vt_b = lane * NSEG + 256
            half = NSEG // 2
            ones_i = jnp.ones((16,), jnp.int32)

            @plsc.parallel_loop(0, 512, unroll=4)
            def _zh(i):
                hist[pl.ds(i * 16, 16)] = jnp.zeros((16,), jnp.int32)

            for p in range(4):
                kc, vc, kn, vn = (ka, va, kb, vb) if p % 2 == 0 else (kb, vb, ka, va)
                shift = p * 8

                @plsc.parallel_loop(0, half, unroll=2)
                def _cnt(g, kc=kc, shift=shift):
                    gia = lane_seg + g
                    gib = gia + half
                    ka_ = plsc.load_gather(kc, [gia])
                    kb_ = plsc.load_gather(kc, [gib])
                    da = lax.shift_right_logical(ka_, shift) & 255
                    db = lax.shift_right_logical(kb_, shift) & 255
                    plsc.addupdate_scatter(hist, [vt_a + da], ones_i)
                    plsc.addupdate_scatter(hist, [vt_b + db], ones_i)

                def b1(c, carry):
                    def acc_l(l, a):
                        return a + hist[pl.ds(l * 256 + c * 16, 16)]

                    tot = lax.fori_loop(0, 32, acc_l, jnp.zeros((16,), jnp.int32))
                    incl = plsc.cumsum(tot)
                    startb[pl.ds(c * 16, 16)] = incl - tot + carry
                    return carry + jnp.sum(tot)

                lax.fori_loop(0, 16, b1, jnp.int32(0))

                def b2(c, _):
                    def bl(l, run):
                        sl2 = pl.ds(l * 256 + c * 16, 16)
                        base[sl2] = run
                        nxt = run + hist[sl2]
                        hist[sl2] = jnp.zeros((16,), jnp.int32)
                        return nxt

                    lax.fori_loop(0, 32, bl, startb[pl.ds(c * 16, 16)])
                    return 0

                lax.fori_loop(0, 16, b2, 0)

                def perm(g, _, p=p, kc=kc, vc=vc, kn=kn, vn=vn, shift=shift):
                    gia = lane_seg + g
                    gib = gia + half
                    ka_ = plsc.load_gather(kc, [gia])
                    kb_ = plsc.load_gather(kc, [gib])
                    va_ = gia if p == 0 else plsc.load_gather(vc, [gia])
                    vb_ = gib if p == 0 else plsc.load_gather(vc, [gib])
                    da = lax.shift_right_logical(ka_, shift) & 255
                    db = lax.shift_right_logical(kb_, shift) & 255
                    ha = vt_a + da
                    hb = vt_b + db
                    pa = plsc.load_gather(base, [ha])
                    plsc.store_scatter(base, [ha], pa + ones_i)
                    pb = plsc.load_gather(base, [hb])
                    plsc.store_scatter(base, [hb], pb + ones_i)
                    plsc.store_scatter(kn, [pa], ka_)
                    plsc.store_scatter(vn, [pa], va_)
                    plsc.store_scatter(kn, [pb], kb_)
                    plsc.store_scatter(vn, [pb], vb_)
                    return 0

                lax.fori_loop(0, half, perm, 0)

            pltpu.sync_copy(va, order_hbm.at[row])
            pltpu.sync_copy(ka, skey_hbm.at[row])

            def zm(g, _):
                maskv[pl.ds(g * 16, 16)] = jnp.zeros((16,), jnp.float32)
                return 0

            lax.fori_loop(0, N // 16, zm, 0)
            ones_f = jnp.ones((16,), jnp.float32)

            def sm(g, _):
                idxv = va[pl.ds(g * 16, 16)]
                plsc.store_scatter(maskv, [idxv], ones_f)
                return 0

            lax.fori_loop(0, NK // 16, sm, 0)
            pltpu.sync_copy(maskv, mask_hbm.at[row])

    return sortk(ikey)


def _sc_select_gather(tokens_flat, order_flat, B, N):
    BN, C = tokens_flat.shape
    NK = N // 2
    NKW = NK // 4
    CHUNK = 64
    mesh = plsc.VectorSubcoreMesh(core_axis_name="c", subcore_axis_name="s")

    @functools.partial(
        pl.kernel,
        out_type=jax.ShapeDtypeStruct((B * NK, C), jnp.float32),
        mesh=mesh,
        scratch_types=[
            pltpu.VMEM((NKW,), jnp.int32),
            pltpu.VMEM((CHUNK, C), jnp.float32),
            pltpu.VMEM((CHUNK, C), jnp.float32),
            pltpu.SemaphoreType.DMA,
            pltpu.SemaphoreType.DMA,
        ],
    )
    def gk(tok_hbm, ord_hbm, out_hbm, idxv, buf0, buf1, sem0, sem1):
        wid = lax.axis_index("s") * 2 + lax.axis_index("c")
        b = wid // 4
        qq = wid % 4
        src_off = b * N + qq * NKW
        dst_off = b * NK + qq * NKW
        pltpu.sync_copy(ord_hbm.at[pl.ds(src_off, NKW)], idxv)
        badd = b * N

        def addb(i, _):
            sl = pl.ds(i * 16, 16)
            idxv[sl] = idxv[sl] + badd
            return 0

        lax.fori_loop(0, NKW // 16, addb, 0)
        bufs = (buf0, buf1)
        sems = (sem0, sem1)

        def start(c):
            return pltpu.async_copy(
                tok_hbm.at[idxv.at[pl.ds(c * CHUNK, CHUNK)]],
                bufs[c % 2], sems[c % 2])

        nch = NKW // CHUNK
        handles = [start(0), None]
        for c in range(nch):
            if c + 1 < nch:
                handles[(c + 1) % 2] = start(c + 1)
            handles[c % 2].wait()
            pltpu.sync_copy(bufs[c % 2],
                            out_hbm.at[pl.ds(dst_off + c * CHUNK, CHUNK)])

    return gk(tokens_flat, order_flat)


def _extra_body(x_ref, k_ref, m_ref, mxk_ref, extra_ref, acc_ref, z_ref):
    kk = k_ref[0, 0]
    ukey = ~kk
    u = jnp.where(ukey < 0, ukey ^ jnp.int32(-2147483648), ~ukey)
    s = lax.bitcast_convert_type(u, jnp.float32)
    mk = ~mxk_ref[0]
    umx = jnp.where(mk < 0, mk ^ jnp.int32(-2147483648), ~mk)
    mx = lax.bitcast_convert_type(umx, jnp.float32)
    e = jnp.exp(s - mx)
    w = jnp.where(m_ref[0, 0] > 0, jnp.float32(0.0), e)
    blk = w.shape[1]
    seg = jnp.arange(blk, dtype=jnp.int32)[None, :] // (blk // 8)
    w8 = jnp.where(seg == jnp.arange(8, dtype=jnp.int32)[:, None], w, jnp.float32(0.0))
    p8 = jnp.dot(w8, x_ref[0])
    p = jnp.sum(p8, axis=0, keepdims=True)
    zp = jnp.sum(w, axis=1, keepdims=True)
    n = pl.program_id(1)

    @pl.when(n == 0)
    def _():
        acc_ref[...] = jnp.zeros_like(acc_ref)
        z_ref[...] = jnp.zeros_like(z_ref)

    acc = acc_ref[...] + p
    z = z_ref[...] + zp
    acc_ref[...] = acc
    z_ref[...] = z

    @pl.when(n == pl.num_programs(1) - 1)
    def _():
        extra_ref[0] = acc / z


def _keymake_body(sp_ref, ac_ref, out_ref):
    score = jnp.float32(0.5) * sp_ref[...] + jnp.float32(0.25) * ac_ref[...]
    u = lax.bitcast_convert_type(score, jnp.int32)
    m32 = (u >> 31) | jnp.int32(-2147483648)
    out_ref[...] = ~(u ^ m32)


def _keyinv_body(k_ref, out_ref):
    ukey = ~k_ref[...]
    u = jnp.where(ukey < 0, ukey ^ jnp.int32(-2147483648), ~ukey)
    out_ref[...] = lax.bitcast_convert_type(u, jnp.float32)


def kernel(tokens, self_attention, cross_attention_m2, cross_attention_m3, W1, b1, W2, b2):
    B, N, C = tokens.shape
    num_keep = N // 2

    acomb = pl.pallas_call(
        _attn_combine_body,
        out_shape=jax.ShapeDtypeStruct((B, N), jnp.float32),
    )(self_attention, cross_attention_m2, cross_attention_m3)

    s_pred3 = pl.pallas_call(
        _mlp_body,
        grid=(B, N // BLK),
        in_specs=[
            pl.BlockSpec((1, BLK, EMBED), lambda b, n: (b, n, 0)),
            pl.BlockSpec((EMBED, HID), lambda b, n: (0, 0)),
            pl.BlockSpec((1, HID), lambda b, n: (0, 0)),
            pl.BlockSpec((HID, 1), lambda b, n: (0, 0)),
            pl.BlockSpec((1, 1), lambda b, n: (0, 0)),
        ],
        out_specs=pl.BlockSpec((1, BLK, 1), lambda b, n: (b, n, 0)),
        out_shape=jax.ShapeDtypeStruct((B, N, 1), jnp.float32),
    )(tokens, W1, b1.reshape(1, HID), W2, b2.reshape(1, 1))

    ikey = pl.pallas_call(
        _keymake_body,
        out_shape=jax.ShapeDtypeStruct((B, N), jnp.int32),
    )(s_pred3.reshape(B, N), acomb)
    order, skeys, score_mask = _sc_sort(ikey)
    keep_policy = order[:, :num_keep]

    select_flat = _sc_select_gather(tokens.reshape(B * N, C), order.reshape(B * N), B, N)
    select_tokens = select_flat.reshape(B, num_keep, C)

    extra = pl.pallas_call(
        _extra_body,
        grid=(B, N // BLK),
        in_specs=[
            pl.BlockSpec((1, BLK, EMBED), lambda b, n: (b, n, 0)),
            pl.BlockSpec((1, 1, 1, BLK), lambda b, n: (b, n, 0, 0)),
            pl.BlockSpec((1, 1, 1, BLK), lambda b, n: (b, n, 0, 0)),
            pl.BlockSpec((1, 1, 1), lambda b, n: (b, 0, 0)),
        ],
        out_specs=pl.BlockSpec((1, 1, EMBED), lambda b, n: (b, 0, 0)),
        out_shape=jax.ShapeDtypeStruct((B, 1, EMBED), jnp.float32),
        scratch_shapes=[
            pltpu.VMEM((1, EMBED), jnp.float32),
            pltpu.VMEM((1, 1), jnp.float32),
        ],
    )(tokens, ikey.reshape(B, N // BLK, 1, BLK), score_mask.reshape(B, N // BLK, 1, BLK),
      skeys[:, num_keep:num_keep + 1].reshape(B, 1, 1))
    extra_token = extra

    selected_mask = jnp.ones((B, num_keep), jnp.float32)
    return (select_tokens, extra_token, score_mask, selected_mask, keep_policy)
